# baseline (speedup 1.0000x reference)
import jax
import jax.numpy as jnp
from jax import lax
from jax.experimental import pallas as pl
from jax.experimental.pallas import tpu as pltpu
from jax.experimental.pallas import tpu_sc as plsc

VOCAB = 1000000
DIM = 32
B = 4096
L = 200
NW = 32
ROWS_PER_W = B // NW
BSUB = 16
LG = 40
NBG = ROWS_PER_W // BSUB
NLG = L // LG
NBLOCKS = NBG * NLG
NPAIRS = NBLOCKS // 2
SCAN_OFFS = (0, 16, 24)


def _sc_body(x_hbm, table_hbm, p1_hbm, p2_hbm, p3_hbm, out_hbm,
             idx_v, rows_a, rows_b, ot_a, ot_b, pref_v, sem_a, sem_b):
    wid = lax.axis_index("s") * 2 + lax.axis_index("c")
    b0w = wid * ROWS_PER_W

    pltpu.sync_copy(p3_hbm, pref_v.at[pl.ds(0, 1)])
    pltpu.sync_copy(p1_hbm, pref_v.at[pl.ds(1, 1)])
    pltpu.sync_copy(p2_hbm, pref_v.at[pl.ds(2, 1)])

    pltpu.sync_copy(x_hbm.at[pl.ds(b0w, ROWS_PER_W)], idx_v)

    lanes = lax.iota(jnp.int32, 16)
    lanes_hi = lanes + 16

    def block_coords(t):
        g = t // NLG
        l0 = pl.multiple_of((t % NLG) * LG, 8)
        return g, l0

    def fire(t, rows_ref, sem):
        g, l0 = block_coords(t)
        for i in range(BSUB):
            pltpu.async_copy(
                table_hbm.at[idx_v.at[g * BSUB + i, pl.ds(l0, LG)]],
                rows_ref.at[i], sem)

    def drain(rows_ref, sem):
        for i in range(BSUB):
            pltpu.make_async_copy(
                table_hbm.at[pl.ds(0, LG)], rows_ref.at[i], sem).wait()

    def process(t, rows_ref, ot_ref):
        g, l0 = block_coords(t)
        any_vec = jnp.zeros((16,), jnp.int32)
        for i in range(BSUB):
            for off in SCAN_OFFS:
                ids = idx_v[g * BSUB + i, pl.ds(l0 + off, 16)]
                u = ids - 4
                m = (u >= 0) & (u < 7)
                any_vec = any_vec | m.astype(jnp.int32)
        block_any = lax.reduce_max(any_vec, (0,))

        @pl.when(block_any > 0)
        def _fixup():
            for i in range(BSUB):
                iv = jnp.full((16,), i, jnp.int32)
                for off in SCAN_OFFS:
                    ids = idx_v[g * BSUB + i, pl.ds(l0 + off, 16)]
                    u = ids - 4
                    m = (u >= 0) & (u < 7)
                    vreg_any = lax.reduce_max(m.astype(jnp.int32), (0,))

                    @pl.when(vreg_any > 0)
                    def _patch():
                        sel = jnp.where(
                            ids == 9, 1,
                            jnp.where(ids == 10, 2, 0)).astype(jnp.int32)
                        pos = off + lanes

                        def dim_step(d, acc):
                            dv = jnp.full((16,), d, jnp.int32)
                            vals = plsc.load_gather(pref_v, [sel, dv])
                            plsc.store_scatter(rows_ref, [iv, pos, dv],
                                               vals, mask=m)
                            return acc

                        lax.fori_loop(0, DIM, dim_step, 0)

        def t_step(lc, acc):
            lcv = jnp.full((16,), lc, jnp.int32)
            for d in range(DIM):
                dv = jnp.full((16,), d, jnp.int32)
                v = plsc.load_gather(rows_ref, [lanes, lcv, dv])
                ot_ref[lc, d, :] = v
            return acc

        lax.fori_loop(0, LG, t_step, 0)

        pltpu.sync_copy(
            ot_ref,
            out_hbm.at[pl.ds(l0, LG), :, pl.ds(b0w + g * BSUB, BSUB)])

    fire(0, rows_a, sem_a)

    def pair_step(p, carry):
        ta = 2 * p
        fire(ta + 1, rows_b, sem_b)
        drain(rows_a, sem_a)
        process(ta, rows_a, ot_a)

        @pl.when(p + 1 < NPAIRS)
        def _prefetch_a():
            fire(ta + 2, rows_a, sem_a)

        drain(rows_b, sem_b)
        process(ta + 1, rows_b, ot_b)
        return carry

    lax.fori_loop(0, NPAIRS, pair_step, 0)


def kernel(x, table, prefix_1, prefix_2, prefix_3):
    run = pl.kernel(
        _sc_body,
        out_type=jax.ShapeDtypeStruct((L, DIM, B), jnp.float32),
        mesh=plsc.VectorSubcoreMesh(core_axis_name="c", subcore_axis_name="s"),
        scratch_types=[
            pltpu.VMEM((ROWS_PER_W, L), jnp.int32),
            pltpu.VMEM((BSUB, LG, DIM), jnp.float32),
            pltpu.VMEM((BSUB, LG, DIM), jnp.float32),
            pltpu.VMEM((LG, DIM, BSUB), jnp.float32),
            pltpu.VMEM((LG, DIM, BSUB), jnp.float32),
            pltpu.VMEM((3, DIM), jnp.float32),
            pltpu.SemaphoreType.DMA,
            pltpu.SemaphoreType.DMA,
        ],
        compiler_params=pltpu.CompilerParams(
            use_tc_tiling_on_sc=False, needs_layout_passes=False),
    )
    ot = run(x.astype(jnp.int32), table, prefix_1, prefix_2, prefix_3)
    return jnp.transpose(ot, (2, 0, 1))


# speedup vs baseline: 1.4318x; 1.4318x over previous
import jax
import jax.numpy as jnp
from jax import lax
from jax.experimental import pallas as pl
from jax.experimental.pallas import tpu as pltpu
from jax.experimental.pallas import tpu_sc as plsc

VOCAB = 1000000
DIM = 32
B = 4096
L = 200
NW = 32
ROWS_PER_W = B // NW
BSUB = 16
LG = 40
NBG = ROWS_PER_W // BSUB
NLG = L // LG
NBLOCKS = NBG * NLG
NPAIRS = NBLOCKS // 2
SCAN_OFFS = (0, 16, 24)


def _sc_body(x_hbm, table_hbm, p1_hbm, p2_hbm, p3_hbm, out_hbm,
             idx_v, rows_a, rows_b, ot_a, ot_b, pref_v, sem_a, sem_b):
    wid = lax.axis_index("s") * 2 + lax.axis_index("c")
    b0w = wid * ROWS_PER_W

    pltpu.sync_copy(p3_hbm, pref_v.at[pl.ds(0, 1)])
    pltpu.sync_copy(p1_hbm, pref_v.at[pl.ds(1, 1)])
    pltpu.sync_copy(p2_hbm, pref_v.at[pl.ds(2, 1)])

    pltpu.sync_copy(x_hbm.at[pl.ds(b0w, ROWS_PER_W)], idx_v)

    lanes = lax.iota(jnp.int32, 16)
    lanes_hi = lanes + 16

    def block_coords(t):
        g = t // NLG
        l0 = pl.multiple_of((t % NLG) * LG, 8)
        return g, l0

    def fire(t, rows_ref, sem):
        g, l0 = block_coords(t)
        for i in range(BSUB):
            pltpu.async_copy(
                table_hbm.at[idx_v.at[g * BSUB + i, pl.ds(l0, LG)]],
                rows_ref.at[i], sem)

    def drain(rows_ref, sem):
        for i in range(BSUB):
            pltpu.make_async_copy(
                table_hbm.at[pl.ds(0, LG)], rows_ref.at[i], sem).wait()

    def process(t, rows_ref, ot_ref):
        g, l0 = block_coords(t)
        any_vec = jnp.zeros((16,), jnp.int32)
        for i in range(BSUB):
            for off in SCAN_OFFS:
                ids = idx_v[g * BSUB + i, pl.ds(l0 + off, 16)]
                u = ids - 4
                m = (u >= 0) & (u < 7)
                any_vec = any_vec | m.astype(jnp.int32)
        block_any = lax.reduce_max(any_vec, (0,))

        @pl.when(block_any > 0)
        def _fixup():
            for i in range(BSUB):
                iv = jnp.full((16,), i, jnp.int32)
                for off in SCAN_OFFS:
                    ids = idx_v[g * BSUB + i, pl.ds(l0 + off, 16)]
                    u = ids - 4
                    m = (u >= 0) & (u < 7)
                    vreg_any = lax.reduce_max(m.astype(jnp.int32), (0,))

                    @pl.when(vreg_any > 0)
                    def _patch():
                        sel = jnp.where(
                            ids == 9, 1,
                            jnp.where(ids == 10, 2, 0)).astype(jnp.int32)
                        pos = off + lanes

                        def dim_step(d, acc):
                            dv = jnp.full((16,), d, jnp.int32)
                            vals = plsc.load_gather(pref_v, [sel, dv])
                            plsc.store_scatter(rows_ref, [iv, pos, dv],
                                               vals, mask=m)
                            return acc

                        lax.fori_loop(0, DIM, dim_step, 0)

        def t_step(lc, acc):
            lcv = jnp.full((16,), lc, jnp.int32)
            for i in range(BSUB):
                iv = jnp.full((16,), i, jnp.int32)
                v0 = rows_ref[i, lc, pl.ds(0, 16)]
                v1 = rows_ref[i, lc, pl.ds(16, 16)]
                plsc.store_scatter(ot_ref, [lcv, lanes, iv], v0)
                plsc.store_scatter(ot_ref, [lcv, lanes_hi, iv], v1)
            return acc

        lax.fori_loop(0, LG, t_step, 0)

        pltpu.sync_copy(
            ot_ref.at[:, :, pl.ds(0, BSUB)],
            out_hbm.at[pl.ds(l0, LG), :, pl.ds(b0w + g * BSUB, BSUB)])

    fire(0, rows_a, sem_a)

    def pair_step(p, carry):
        ta = 2 * p
        fire(ta + 1, rows_b, sem_b)
        drain(rows_a, sem_a)
        process(ta, rows_a, ot_a)

        @pl.when(p + 1 < NPAIRS)
        def _prefetch_a():
            fire(ta + 2, rows_a, sem_a)

        drain(rows_b, sem_b)
        process(ta + 1, rows_b, ot_b)
        return carry

    lax.fori_loop(0, NPAIRS, pair_step, 0)


def kernel(x, table, prefix_1, prefix_2, prefix_3):
    run = pl.kernel(
        _sc_body,
        out_type=jax.ShapeDtypeStruct((L, DIM, B), jnp.float32),
        mesh=plsc.VectorSubcoreMesh(core_axis_name="c", subcore_axis_name="s"),
        scratch_types=[
            pltpu.VMEM((ROWS_PER_W, L), jnp.int32),
            pltpu.VMEM((BSUB, LG, DIM), jnp.float32),
            pltpu.VMEM((BSUB, LG, DIM), jnp.float32),
            pltpu.VMEM((LG, DIM, BSUB + 1), jnp.float32),
            pltpu.VMEM((LG, DIM, BSUB + 1), jnp.float32),
            pltpu.VMEM((3, DIM), jnp.float32),
            pltpu.SemaphoreType.DMA,
            pltpu.SemaphoreType.DMA,
        ],
        compiler_params=pltpu.CompilerParams(
            use_tc_tiling_on_sc=False, needs_layout_passes=False),
    )
    ot = run(x.astype(jnp.int32), table, prefix_1, prefix_2, prefix_3)
    return jnp.transpose(ot, (2, 0, 1))
